# baseline (speedup 1.0000x reference)
import functools

import jax
import jax.numpy as jnp
from jax import lax
from jax.experimental import pallas as pl
from jax.experimental.pallas import tpu as pltpu
from jax.experimental.pallas import tpu_sc as plsc

_N = 1048576
_NW = 32
_VR = _N * 4 // 128
_VRW = _VR // _NW
_CR = 128
_NCHUNK = _VRW // _CR
_LCH = _CR * 32


def _sc_partials(a2d, b2d, lab_flat):
    mesh = plsc.VectorSubcoreMesh(core_axis_name="c", subcore_axis_name="s")

    @functools.partial(
        pl.kernel,
        mesh=mesh,
        out_type=[
            jax.ShapeDtypeStruct((_NW * 16,), jnp.float32),
            jax.ShapeDtypeStruct((_NW * 16,), jnp.float32),
        ],
        scratch_types=[
            pltpu.VMEM((_CR, 128), jnp.float32),
            pltpu.VMEM((_CR, 128), jnp.float32),
            pltpu.VMEM((_LCH,), jnp.float32),
            pltpu.VMEM((16,), jnp.float32),
            pltpu.VMEM((16,), jnp.float32),
        ],
    )
    def k(a_hbm, b_hbm, lab_hbm, acc_out, cnt_out, a_v, b_v, l_v, acc_v, cnt_v):
        wid = lax.axis_index("s") * 2 + lax.axis_index("c")
        vbase = wid * _VRW
        lane = lax.iota(jnp.int32, 16)
        sub = lane >> 2

        def chunk_body(ci, carry):
            acc, cnt = carry
            v0 = vbase + ci * _CR
            pltpu.sync_copy(a_hbm.at[pl.ds(v0, _CR)], a_v)
            pltpu.sync_copy(b_hbm.at[pl.ds(v0, _CR)], b_v)
            pltpu.sync_copy(lab_hbm.at[pl.ds(v0 * 32, _LCH)], l_v)

            def g_body(vr, carry2):
                acc, cnt = carry2
                for h in range(2):
                    lab = l_v[pl.ds(vr * 32 + h * 16, 16)]
                    m = lab * lab
                    cnt = cnt + m
                    for c in range(4):
                        av = a_v[vr, pl.ds(h * 64 + c * 16, 16)]
                        bv = b_v[vr, pl.ds(h * 64 + c * 16, 16)]
                        d = av - bv
                        mm = m.at[sub + 4 * c].get(mode="promise_in_bounds")
                        acc = acc + d * d * mm
                return acc, cnt

            return lax.fori_loop(0, _CR, g_body, (acc, cnt))

        acc0 = jnp.zeros((16,), jnp.float32)
        cnt0 = jnp.zeros((16,), jnp.float32)
        acc, cnt = lax.fori_loop(0, _NCHUNK, chunk_body, (acc0, cnt0))
        acc_v[...] = acc
        cnt_v[...] = cnt
        pltpu.sync_copy(acc_v, acc_out.at[pl.ds(wid * 16, 16)])
        pltpu.sync_copy(cnt_v, cnt_out.at[pl.ds(wid * 16, 16)])

    return k(a2d, b2d, lab_flat)


def kernel(bbox_out, bbox_target, label):
    a = bbox_out.reshape(_VR, 128)
    b = bbox_target.reshape(_VR, 128)
    lab = label.reshape(-1)
    acc, cnt = _sc_partials(a, b, lab)
    total = jnp.sum(acc)
    keep_num = jnp.sum(cnt)
    return total / keep_num
